# Initial kernel scaffold; baseline (speedup 1.0000x reference)
import jax
import jax.numpy as jnp
from jax.experimental import pallas as pl


def kernel(x, W_embed, b_embed, W_gate, Wi, bi, Wo, bo, W_proj, b_proj):
    raise NotImplementedError("write your pallas kernel here")


# speedup vs baseline: 10.0193x; 10.0193x over previous
import functools

import jax
import jax.numpy as jnp
from jax.experimental import pallas as pl

T = 32768
D_IN = 4
D_HID = 16
D_FF = 32
E = 8

BLK = 2048


def _rne_bf16(a):
    u = jax.lax.bitcast_convert_type(a, jnp.uint32)
    r = (u + jnp.uint32(0x7FFF) + ((u >> jnp.uint32(16)) & jnp.uint32(1)))
    r = r & jnp.uint32(0xFFFF0000)
    return jax.lax.bitcast_convert_type(r, jnp.float32)


def _moe_kernel(x_ref, we_ref, be_ref, wg_ref, wi_ref, bi_ref, wo_ref,
                bo_ref, wp_ref, bp_ref, out_ref):
    x = _rne_bf16(x_ref[:, :])

    we = _rne_bf16(we_ref[:, :])
    h = we[:, 0:1] * x[0:1, :]
    for k in range(1, D_IN):
        h = h + we[:, k:k + 1] * x[k:k + 1, :]
    h = h + be_ref[:, :]

    hb = _rne_bf16(h)
    wg = _rne_bf16(wg_ref[:, :])
    logits = wg[:, 0:1] * hb[0:1, :]
    for k in range(1, D_HID):
        logits = logits + wg[:, k:k + 1] * hb[k:k + 1, :]

    m = jnp.max(logits, axis=0, keepdims=True)
    s = jnp.sum(jnp.exp(logits - m), axis=0, keepdims=True)
    gate = 1.0 / s
    rows = jax.lax.broadcasted_iota(jnp.int32, (E, BLK), 0)
    idx = jnp.min(jnp.where(logits == m, rows, E), axis=0, keepdims=True)

    acc = None
    for e in range(E):
        mid = wi_ref[e, :, 0:1] * h[0:1, :]
        for k in range(1, D_HID):
            mid = mid + wi_ref[e, :, k:k + 1] * h[k:k + 1, :]
        mid = jax.nn.gelu(mid + bi_ref[:, e:e + 1])
        o = wo_ref[e, :, 0:1] * mid[0:1, :]
        for f in range(1, D_FF):
            o = o + wo_ref[e, :, f:f + 1] * mid[f:f + 1, :]
        o = o + bo_ref[:, e:e + 1]
        w_e = jnp.where(idx == e, gate, 0.0)
        acc = w_e * o if acc is None else acc + w_e * o

    out = wp_ref[:, 0:1] * acc[0:1, :]
    for k in range(1, D_HID):
        out = out + wp_ref[:, k:k + 1] * acc[k:k + 1, :]
    out_ref[:, :] = out + bp_ref[:, :]


@functools.partial(jax.jit, static_argnames=())
def kernel(x, W_embed, b_embed, W_gate, Wi, bi, Wo, bo, W_proj, b_proj):
    xT = x.T
    weT = W_embed.T
    be2 = b_embed[:, None]
    wgT = W_gate.T
    wiT = Wi.transpose(0, 2, 1)
    biT = bi.T
    woT = Wo.transpose(0, 2, 1)
    boT = bo.T
    wpT = W_proj.T
    bp2 = b_proj[:, None]

    grid = (T // BLK,)
    full = lambda shape: pl.BlockSpec(shape, lambda i: tuple(0 for _ in shape))
    outT = pl.pallas_call(
        _moe_kernel,
        grid=grid,
        in_specs=[
            pl.BlockSpec((D_IN, BLK), lambda i: (0, i)),
            full((D_HID, D_IN)),
            full((D_HID, 1)),
            full((E, D_HID)),
            full((E, D_FF, D_HID)),
            full((D_FF, E)),
            full((E, D_HID, D_FF)),
            full((D_HID, E)),
            full((D_IN, D_HID)),
            full((D_IN, 1)),
        ],
        out_specs=pl.BlockSpec((D_IN, BLK), lambda i: (0, i)),
        out_shape=jax.ShapeDtypeStruct((D_IN, T), jnp.float32),
    )(xT, weT, be2, wgT, wiT, biT, woT, boT, wpT, bp2)
    return outT.T


# speedup vs baseline: 18.1950x; 1.8160x over previous
import functools

import jax
import jax.numpy as jnp
from jax.experimental import pallas as pl

T = 32768
D_IN = 4
D_HID = 16
D_FF = 32
E = 8
EF = E * D_FF

BLK = 2048

f32 = jnp.float32
bf16 = jnp.bfloat16


def _moe_kernel(x_ref, we_ref, be_ref, wg_ref, wi_ref, bi_ref, wo_ref,
                bo_ref, wp_ref, bp_ref, out_ref):
    dot = functools.partial(jax.lax.dot_general,
                            preferred_element_type=f32)
    dims = (((1,), (0,)), ((), ()))

    h = dot(x_ref[:, :], we_ref[:, :], dims) + be_ref[:, :]
    hb = h.astype(bf16)

    logits = dot(hb, wg_ref[:, :], dims)

    m = jnp.max(logits, axis=1, keepdims=True)
    s = jnp.sum(jnp.exp(logits - m), axis=1, keepdims=True)
    gate = 1.0 / s
    lanes = jax.lax.broadcasted_iota(jnp.int32, (BLK, E), 1)
    idx = jnp.min(jnp.where(logits == m, lanes, E), axis=1, keepdims=True)

    mid = jax.nn.gelu(dot(hb, wi_ref[:, :], dims) + bi_ref[:, :])
    col_e = jax.lax.broadcasted_iota(jnp.int32, (BLK, EF), 1) // D_FF
    mmask = jnp.where(col_e == idx, mid, 0.0).astype(bf16)

    o = dot(mmask, wo_ref[:, :], dims)
    bo_sel = jnp.zeros((BLK, D_HID), f32)
    for e in range(E):
        sel = (idx == e).astype(f32)
        bo_sel = bo_sel + sel * bo_ref[e:e + 1, :]
    moe = (o + bo_sel) * gate

    out_ref[:, :] = dot(moe.astype(bf16), wp_ref[:, :], dims) + bp_ref[:, :]


@jax.jit
def kernel(x, W_embed, b_embed, W_gate, Wi, bi, Wo, bo, W_proj, b_proj):
    xb = x.astype(bf16)
    web = W_embed.astype(bf16)
    be2 = b_embed[None, :]
    wgb = W_gate.astype(bf16)
    wi_flat = Wi.transpose(1, 0, 2).reshape(D_HID, EF).astype(bf16)
    bi_flat = bi.reshape(1, EF)
    wo_flat = Wo.reshape(EF, D_HID).astype(bf16)
    wpb = W_proj.astype(bf16)
    bp2 = b_proj[None, :]

    grid = (T // BLK,)
    full = lambda shape: pl.BlockSpec(shape, lambda i: tuple(0 for _ in shape))
    return pl.pallas_call(
        _moe_kernel,
        grid=grid,
        in_specs=[
            pl.BlockSpec((BLK, D_IN), lambda i: (i, 0)),
            full((D_IN, D_HID)),
            full((1, D_HID)),
            full((D_HID, E)),
            full((D_HID, EF)),
            full((1, EF)),
            full((EF, D_HID)),
            full((E, D_HID)),
            full((D_HID, D_IN)),
            full((1, D_IN)),
        ],
        out_specs=pl.BlockSpec((BLK, D_IN), lambda i: (i, 0)),
        out_shape=jax.ShapeDtypeStruct((T, D_IN), f32),
    )(xb, web, be2, wgb, wi_flat, bi_flat, wo_flat, bo, wpb, bp2)
